# speedup vs baseline: 1.1386x; 1.1386x over previous
import functools

import jax
import jax.numpy as jnp
from jax.experimental import pallas as pl

_NEG_SLOPE = 0.2


def _matmul2(x, Wl, Wr, n_blocks=10):
    N, F = x.shape
    C = Wl.shape[1]

    def body(x_ref, wl_ref, wr_ref, l_ref, r_ref):
        l_ref[...] = x_ref[...] @ wl_ref[...]
        r_ref[...] = x_ref[...] @ wr_ref[...]

    return pl.pallas_call(
        body,
        out_shape=(jax.ShapeDtypeStruct((N, C), x.dtype),
                   jax.ShapeDtypeStruct((N, C), x.dtype)),
        grid=(n_blocks,),
        in_specs=[
            pl.BlockSpec((N // n_blocks, F), lambda i: (i, 0)),
            pl.BlockSpec((F, C), lambda i: (0, 0)),
            pl.BlockSpec((F, C), lambda i: (0, 0)),
        ],
        out_specs=(
            pl.BlockSpec((N // n_blocks, C), lambda i: (i, 0)),
            pl.BlockSpec((N // n_blocks, C), lambda i: (i, 0)),
        ),
    )(x, Wl, Wr)


def _gat_layer(x, src, dst, Wl, Wr, att, bias, num_nodes):
    l, r = _matmul2(x, Wl, Wr)
    z = jax.nn.leaky_relu(l[src] + r[dst], negative_slope=_NEG_SLOPE)
    e = (z * att).sum(axis=-1)
    e_max = jax.ops.segment_max(e, dst, num_segments=num_nodes)
    e_exp = jnp.exp(e - e_max[dst])
    denom = jax.ops.segment_sum(e_exp, dst, num_segments=num_nodes)
    alpha = e_exp / (denom[dst] + 1e-16)
    out = jax.ops.segment_sum(alpha[:, None] * l[src], dst, num_segments=num_nodes)
    return out + bias


def kernel(X, ei_feat, batch, Wl1, Wr1, att1, b1, Wl2, Wr2, att2, b2):
    num_nodes = X.shape[0]
    loop = jnp.arange(num_nodes, dtype=ei_feat.dtype)
    src = jnp.concatenate([ei_feat[0], loop])
    dst = jnp.concatenate([ei_feat[1], loop])
    h = jax.nn.elu(_gat_layer(X, src, dst, Wl1, Wr1, att1, b1, num_nodes))
    out = _gat_layer(h, src, dst, Wl2, Wr2, att2, b2, num_nodes)
    return jax.nn.softmax(out, axis=1)


# speedup vs baseline: 3.8682x; 3.3973x over previous
import functools

import jax
import jax.numpy as jnp
from jax import lax
from jax.experimental import pallas as pl
from jax.experimental.pallas import tpu as pltpu
from jax.experimental.pallas import tpu_sc as plsc

_N = 10000
_NEG = 0.2
_NC = 2
_NS = 16
_L = 16
_NW = _NC * _NS
_G = 64
_E = 330000
_GPW = -(-_E // (_NW * _G))
_EPAD = _NW * _GPW * _G
_NPAD = 10240
_RPT = _NPAD // _NS
_RCH = _RPT // _G


def _sc_gat_edges(l, r_pad, src, dst, att, C):
    CW = C + 16
    mesh = plsc.VectorSubcoreMesh(core_axis_name="c", subcore_axis_name="s")

    def body(l_hbm, r_hbm, src_hbm, dst_hbm, att_hbm, out_hbm,
             src_v, dst_v, lrows, rrows, wrows, attv, acc_sh, sem_l, sem_r):
        ci = lax.axis_index("c")
        si = lax.axis_index("s")
        wid = ci * _NS + si
        iota = lax.iota(jnp.int32, _L)
        zeros16 = jnp.zeros((_L,), jnp.float32)

        def zrow(i, carry):
            for k in range(CW // _L):
                wrows[i, pl.ds(k * _L, _L)] = zeros16
            return carry
        lax.fori_loop(0, _G, zrow, 0)

        row0 = si * _RPT
        for j in range(_RCH):
            pltpu.sync_copy(wrows, acc_sh.at[pl.ds(row0 + j * _G, _G)])
        pltpu.sync_copy(att_hbm, attv)
        plsc.subcore_barrier()

        def group(gi, carry):
            base = (wid * _GPW + gi) * _G
            pltpu.sync_copy(src_hbm.at[pl.ds(base, _G)], src_v)
            pltpu.sync_copy(dst_hbm.at[pl.ds(base, _G)], dst_v)
            cpl = pltpu.async_copy(l_hbm.at[src_v], lrows, sem_l)
            cpr = pltpu.async_copy(r_hbm.at[dst_v], rrows, sem_r)
            cpl.wait()
            cpr.wait()
            for sub in range(_G // _L):
                eidx = sub * _L + iota

                def dot(cc, acc):
                    cv = jnp.full((_L,), cc, jnp.int32)
                    vl = plsc.load_gather(lrows, [eidx, cv])
                    vr = plsc.load_gather(rrows, [eidx, cv])
                    av = plsc.load_gather(attv, [cv])
                    s = vl + vr
                    z = jnp.maximum(s, 0.0) + _NEG * jnp.minimum(s, 0.0)
                    return acc + av * z

                w = jnp.exp(lax.fori_loop(0, C, dot, zeros16))
                plsc.store_scatter(wrows, [eidx, jnp.full((_L,), C, jnp.int32)], w)

                def wmul(cc, carry2):
                    cv = jnp.full((_L,), cc, jnp.int32)
                    vl = plsc.load_gather(lrows, [eidx, cv])
                    plsc.store_scatter(wrows, [eidx, cv], w * vl)
                    return carry2

                lax.fori_loop(0, C, wmul, 0)
            pltpu.sync_copy(wrows, acc_sh.at[dst_v], add=True)
            return carry
        lax.fori_loop(0, _GPW, group, 0)

        plsc.subcore_barrier()
        for j in range(_RCH):
            sl = pl.ds(row0 + j * _G, _G)
            pltpu.sync_copy(acc_sh.at[sl], out_hbm.at[ci, sl])

    k = pl.kernel(
        body,
        out_type=jax.ShapeDtypeStruct((_NC, _NPAD, CW), jnp.float32),
        mesh=mesh,
        compiler_params=pltpu.CompilerParams(needs_layout_passes=False,
                                             use_tc_tiling_on_sc=False),
        scratch_types=[
            pltpu.VMEM((_G,), jnp.int32),
            pltpu.VMEM((_G,), jnp.int32),
            pltpu.VMEM((_G, C), jnp.float32),
            pltpu.VMEM((_G, C), jnp.float32),
            pltpu.VMEM((_G, CW), jnp.float32),
            pltpu.VMEM((C,), jnp.float32),
            pltpu.VMEM_SHARED((_NPAD, CW), jnp.float32),
            pltpu.SemaphoreType.DMA,
            pltpu.SemaphoreType.DMA,
        ],
    )
    return k(l, r_pad, src, dst, att)


def _matmul2(x, Wl, Wr, nb=10):
    N, F = x.shape
    C = Wl.shape[1]

    def body(x_ref, wl_ref, wr_ref, l_ref, r_ref):
        l_ref[...] = x_ref[...] @ wl_ref[...]
        r_ref[...] = x_ref[...] @ wr_ref[...]

    return pl.pallas_call(
        body,
        out_shape=(jax.ShapeDtypeStruct((N, C), x.dtype),
                   jax.ShapeDtypeStruct((N, C), x.dtype)),
        grid=(nb,),
        in_specs=[
            pl.BlockSpec((N // nb, F), lambda i: (i, 0)),
            pl.BlockSpec((F, C), lambda i: (0, 0)),
            pl.BlockSpec((F, C), lambda i: (0, 0)),
        ],
        out_specs=(
            pl.BlockSpec((N // nb, C), lambda i: (i, 0)),
            pl.BlockSpec((N // nb, C), lambda i: (i, 0)),
        ),
    )(x, Wl, Wr)


def _mid(n0, n1, d0, d1, b, Wl, Wr, nb=10):
    N, C = n0.shape
    DW = d0.shape[1]
    K = Wl.shape[1]

    def body(n0_ref, n1_ref, d0_ref, d1_ref, b_ref, wl_ref, wr_ref, l_ref, r_ref):
        num = n0_ref[...] + n1_ref[...]
        den = jnp.sum(d0_ref[...] + d1_ref[...], axis=1, keepdims=True) + 1e-16
        x = num / den + b_ref[...]
        h = jnp.where(x > 0, x, jnp.exp(jnp.minimum(x, 0.0)) - 1.0)
        l_ref[...] = h @ wl_ref[...]
        r_ref[...] = h @ wr_ref[...]

    return pl.pallas_call(
        body,
        out_shape=(jax.ShapeDtypeStruct((N, K), n0.dtype),
                   jax.ShapeDtypeStruct((N, K), n0.dtype)),
        grid=(nb,),
        in_specs=[
            pl.BlockSpec((N // nb, C), lambda i: (i, 0)),
            pl.BlockSpec((N // nb, C), lambda i: (i, 0)),
            pl.BlockSpec((N // nb, DW), lambda i: (i, 0)),
            pl.BlockSpec((N // nb, DW), lambda i: (i, 0)),
            pl.BlockSpec((1, C), lambda i: (0, 0)),
            pl.BlockSpec((C, K), lambda i: (0, 0)),
            pl.BlockSpec((C, K), lambda i: (0, 0)),
        ],
        out_specs=(
            pl.BlockSpec((N // nb, K), lambda i: (i, 0)),
            pl.BlockSpec((N // nb, K), lambda i: (i, 0)),
        ),
    )(n0, n1, d0, d1, b, Wl, Wr)


def _final(m0, m1, d0, d1, b, nb=10):
    N, K = m0.shape
    DW = d0.shape[1]

    def body(m0_ref, m1_ref, d0_ref, d1_ref, b_ref, o_ref):
        num = m0_ref[...] + m1_ref[...]
        den = jnp.sum(d0_ref[...] + d1_ref[...], axis=1, keepdims=True) + 1e-16
        x = num / den + b_ref[...]
        m = jnp.max(x, axis=1, keepdims=True)
        ez = jnp.exp(x - m)
        o_ref[...] = ez / jnp.sum(ez, axis=1, keepdims=True)

    return pl.pallas_call(
        body,
        out_shape=jax.ShapeDtypeStruct((N, K), m0.dtype),
        grid=(nb,),
        in_specs=[
            pl.BlockSpec((N // nb, K), lambda i: (i, 0)),
            pl.BlockSpec((N // nb, K), lambda i: (i, 0)),
            pl.BlockSpec((N // nb, DW), lambda i: (i, 0)),
            pl.BlockSpec((N // nb, DW), lambda i: (i, 0)),
            pl.BlockSpec((1, K), lambda i: (0, 0)),
        ],
        out_specs=pl.BlockSpec((N // nb, K), lambda i: (i, 0)),
    )(m0, m1, d0, d1, b)


def kernel(X, ei_feat, batch, Wl1, Wr1, att1, b1, Wl2, Wr2, att2, b2):
    N = X.shape[0]
    loop = jnp.arange(N, dtype=jnp.int32)
    npad_e = _EPAD - _E
    src = jnp.concatenate([ei_feat[0].astype(jnp.int32), loop,
                           jnp.zeros((npad_e,), jnp.int32)])
    dst = jnp.concatenate([ei_feat[1].astype(jnp.int32), loop,
                           jnp.full((npad_e,), N, jnp.int32)])

    l1, r1 = _matmul2(X, Wl1, Wr1)
    r1p = jnp.concatenate([r1, jnp.zeros((16, r1.shape[1]), r1.dtype)])
    p1 = _sc_gat_edges(l1, r1p, src, dst, att1, 128)
    l2, r2 = _mid(p1[0, :N, :128], p1[1, :N, :128],
                  p1[0, :N, 128:144], p1[1, :N, 128:144],
                  b1.reshape(1, -1), Wl2, Wr2)

    r2p = jnp.concatenate([r2, jnp.zeros((16, r2.shape[1]), r2.dtype)])
    p2 = _sc_gat_edges(l2, r2p, src, dst, att2, 16)
    return _final(p2[0, :N, :16], p2[1, :N, :16],
                  p2[0, :N, 16:32], p2[1, :N, 16:32],
                  b2.reshape(1, -1))


# speedup vs baseline: 4.1060x; 1.0615x over previous
import functools

import jax
import jax.numpy as jnp
from jax import lax
from jax.experimental import pallas as pl
from jax.experimental.pallas import tpu as pltpu
from jax.experimental.pallas import tpu_sc as plsc

_N = 10000
_NEG = 0.2
_NC = 2
_NS = 16
_L = 16
_NW = _NC * _NS
_G = 64
_E = 330000
_GPW = -(-_E // (_NW * _G))
_EPAD = _NW * _GPW * _G
_NPAD = 10240
_RPT = _NPAD // _NS
_RCH = _RPT // _G


def _sc_gat_edges(l, r_pad, src, dst, att, C):
    CW = C + 16
    mesh = plsc.VectorSubcoreMesh(core_axis_name="c", subcore_axis_name="s")

    def body(l_hbm, r_hbm, src_hbm, dst_hbm, att_hbm, out_hbm,
             src_v, dst_v, lrows, rrows, wrows, attv, acc_sh, sem_l, sem_r):
        ci = lax.axis_index("c")
        si = lax.axis_index("s")
        wid = ci * _NS + si
        iota = lax.iota(jnp.int32, _L)
        zeros16 = jnp.zeros((_L,), jnp.float32)

        def zrow(i, carry):
            for k in range(CW // _L):
                wrows[i, pl.ds(k * _L, _L)] = zeros16
            return carry
        lax.fori_loop(0, _G, zrow, 0)

        row0 = si * _RPT
        for j in range(_RCH):
            pltpu.sync_copy(wrows, acc_sh.at[pl.ds(row0 + j * _G, _G)])
        pltpu.sync_copy(att_hbm, attv)
        plsc.subcore_barrier()

        def group(gi, carry):
            base = (wid * _GPW + gi) * _G
            pltpu.sync_copy(src_hbm.at[pl.ds(base, _G)], src_v)
            pltpu.sync_copy(dst_hbm.at[pl.ds(base, _G)], dst_v)
            cpl = pltpu.async_copy(l_hbm.at[src_v], lrows, sem_l)
            cpr = pltpu.async_copy(r_hbm.at[dst_v], rrows, sem_r)
            cpl.wait()
            cpr.wait()
            U = 16
            for sub in range(_G // _L):
                eidx = sub * _L + iota

                def dot_chunk(cb, acc):
                    for u in range(U):
                        cv = jnp.full((_L,), cb * U + u, jnp.int32)
                        vl = plsc.load_gather(lrows, [eidx, cv])
                        vr = plsc.load_gather(rrows, [eidx, cv])
                        av = plsc.load_gather(attv, [cv])
                        s = vl + vr
                        z = jnp.maximum(s, 0.0) + _NEG * jnp.minimum(s, 0.0)
                        acc = acc + av * z
                    return acc

                if C == U:
                    acc = dot_chunk(0, zeros16)
                else:
                    acc = lax.fori_loop(0, C // U, dot_chunk, zeros16)
                w = jnp.exp(acc)
                plsc.store_scatter(wrows, [eidx, jnp.full((_L,), C, jnp.int32)], w)

                def wmul_chunk(cb, carry2):
                    for u in range(U):
                        cv = jnp.full((_L,), cb * U + u, jnp.int32)
                        vl = plsc.load_gather(lrows, [eidx, cv])
                        plsc.store_scatter(wrows, [eidx, cv], w * vl)
                    return carry2

                if C == U:
                    wmul_chunk(0, 0)
                else:
                    lax.fori_loop(0, C // U, wmul_chunk, 0)
            pltpu.sync_copy(wrows, acc_sh.at[dst_v], add=True)
            return carry
        lax.fori_loop(0, _GPW, group, 0)

        plsc.subcore_barrier()
        for j in range(_RCH):
            sl = pl.ds(row0 + j * _G, _G)
            pltpu.sync_copy(acc_sh.at[sl], out_hbm.at[ci, sl])

    k = pl.kernel(
        body,
        out_type=jax.ShapeDtypeStruct((_NC, _NPAD, CW), jnp.float32),
        mesh=mesh,
        compiler_params=pltpu.CompilerParams(needs_layout_passes=False,
                                             use_tc_tiling_on_sc=False),
        scratch_types=[
            pltpu.VMEM((_G,), jnp.int32),
            pltpu.VMEM((_G,), jnp.int32),
            pltpu.VMEM((_G, C), jnp.float32),
            pltpu.VMEM((_G, C), jnp.float32),
            pltpu.VMEM((_G, CW), jnp.float32),
            pltpu.VMEM((C,), jnp.float32),
            pltpu.VMEM_SHARED((_NPAD, CW), jnp.float32),
            pltpu.SemaphoreType.DMA,
            pltpu.SemaphoreType.DMA,
        ],
    )
    return k(l, r_pad, src, dst, att)


def _matmul2(x, Wl, Wr, nb=10):
    N, F = x.shape
    C = Wl.shape[1]

    def body(x_ref, wl_ref, wr_ref, l_ref, r_ref):
        l_ref[...] = x_ref[...] @ wl_ref[...]
        r_ref[...] = x_ref[...] @ wr_ref[...]

    return pl.pallas_call(
        body,
        out_shape=(jax.ShapeDtypeStruct((N, C), x.dtype),
                   jax.ShapeDtypeStruct((N, C), x.dtype)),
        grid=(nb,),
        in_specs=[
            pl.BlockSpec((N // nb, F), lambda i: (i, 0)),
            pl.BlockSpec((F, C), lambda i: (0, 0)),
            pl.BlockSpec((F, C), lambda i: (0, 0)),
        ],
        out_specs=(
            pl.BlockSpec((N // nb, C), lambda i: (i, 0)),
            pl.BlockSpec((N // nb, C), lambda i: (i, 0)),
        ),
    )(x, Wl, Wr)


def _mid(n0, n1, d0, d1, b, Wl, Wr, nb=10):
    N, C = n0.shape
    DW = d0.shape[1]
    K = Wl.shape[1]

    def body(n0_ref, n1_ref, d0_ref, d1_ref, b_ref, wl_ref, wr_ref, l_ref, r_ref):
        num = n0_ref[...] + n1_ref[...]
        den = jnp.sum(d0_ref[...] + d1_ref[...], axis=1, keepdims=True) + 1e-16
        x = num / den + b_ref[...]
        h = jnp.where(x > 0, x, jnp.exp(jnp.minimum(x, 0.0)) - 1.0)
        l_ref[...] = h @ wl_ref[...]
        r_ref[...] = h @ wr_ref[...]

    return pl.pallas_call(
        body,
        out_shape=(jax.ShapeDtypeStruct((N, K), n0.dtype),
                   jax.ShapeDtypeStruct((N, K), n0.dtype)),
        grid=(nb,),
        in_specs=[
            pl.BlockSpec((N // nb, C), lambda i: (i, 0)),
            pl.BlockSpec((N // nb, C), lambda i: (i, 0)),
            pl.BlockSpec((N // nb, DW), lambda i: (i, 0)),
            pl.BlockSpec((N // nb, DW), lambda i: (i, 0)),
            pl.BlockSpec((1, C), lambda i: (0, 0)),
            pl.BlockSpec((C, K), lambda i: (0, 0)),
            pl.BlockSpec((C, K), lambda i: (0, 0)),
        ],
        out_specs=(
            pl.BlockSpec((N // nb, K), lambda i: (i, 0)),
            pl.BlockSpec((N // nb, K), lambda i: (i, 0)),
        ),
    )(n0, n1, d0, d1, b, Wl, Wr)


def _final(m0, m1, d0, d1, b, nb=10):
    N, K = m0.shape
    DW = d0.shape[1]

    def body(m0_ref, m1_ref, d0_ref, d1_ref, b_ref, o_ref):
        num = m0_ref[...] + m1_ref[...]
        den = jnp.sum(d0_ref[...] + d1_ref[...], axis=1, keepdims=True) + 1e-16
        x = num / den + b_ref[...]
        m = jnp.max(x, axis=1, keepdims=True)
        ez = jnp.exp(x - m)
        o_ref[...] = ez / jnp.sum(ez, axis=1, keepdims=True)

    return pl.pallas_call(
        body,
        out_shape=jax.ShapeDtypeStruct((N, K), m0.dtype),
        grid=(nb,),
        in_specs=[
            pl.BlockSpec((N // nb, K), lambda i: (i, 0)),
            pl.BlockSpec((N // nb, K), lambda i: (i, 0)),
            pl.BlockSpec((N // nb, DW), lambda i: (i, 0)),
            pl.BlockSpec((N // nb, DW), lambda i: (i, 0)),
            pl.BlockSpec((1, K), lambda i: (0, 0)),
        ],
        out_specs=pl.BlockSpec((N // nb, K), lambda i: (i, 0)),
    )(m0, m1, d0, d1, b)


def kernel(X, ei_feat, batch, Wl1, Wr1, att1, b1, Wl2, Wr2, att2, b2):
    N = X.shape[0]
    loop = jnp.arange(N, dtype=jnp.int32)
    npad_e = _EPAD - _E
    src = jnp.concatenate([ei_feat[0].astype(jnp.int32), loop,
                           jnp.zeros((npad_e,), jnp.int32)])
    dst = jnp.concatenate([ei_feat[1].astype(jnp.int32), loop,
                           jnp.full((npad_e,), N, jnp.int32)])

    l1, r1 = _matmul2(X, Wl1, Wr1)
    r1p = jnp.concatenate([r1, jnp.zeros((16, r1.shape[1]), r1.dtype)])
    p1 = _sc_gat_edges(l1, r1p, src, dst, att1, 128)
    l2, r2 = _mid(p1[0, :N, :128], p1[1, :N, :128],
                  p1[0, :N, 128:144], p1[1, :N, 128:144],
                  b1.reshape(1, -1), Wl2, Wr2)

    r2p = jnp.concatenate([r2, jnp.zeros((16, r2.shape[1]), r2.dtype)])
    p2 = _sc_gat_edges(l2, r2p, src, dst, att2, 16)
    return _final(p2[0, :N, :16], p2[1, :N, :16],
                  p2[0, :N, 16:32], p2[1, :N, 16:32],
                  b2.reshape(1, -1))


# speedup vs baseline: 11.8198x; 2.8787x over previous
import functools

import jax
import jax.numpy as jnp
from jax import lax
from jax.experimental import pallas as pl
from jax.experimental.pallas import tpu as pltpu
from jax.experimental.pallas import tpu_sc as plsc

_N = 10000
_NEG = 0.2
_NC = 2
_NS = 16
_L = 16
_NW = _NC * _NS
_G = 48
_E = 330000
_GPW = 224
_EPAD = _NW * _GPW * _G
_EALLOC = _EPAD + 2 * _G
_NPAD = 10016
_RPT = _NPAD // _NS


def _sc_gat_edges(l, r_pad, src, dst, att, C, unroll=4):
    CW = C + 16
    NK = C // _L
    mesh = plsc.VectorSubcoreMesh(core_axis_name="c", subcore_axis_name="s")

    def body(l_hbm, r_hbm, src_hbm, dst_hbm, att_hbm, out_hbm,
             s0, s1, s2, s3, d0, d1, d2, d3,
             lrows, rrows, wrows, attv, acc_sh,
             si0, si1, si2, si3, sgl0, sgl1, sgr0, sgr1, ss0, ss1):
        sv = [s0, s1, s2, s3]
        dv = [d0, d1, d2, d3]
        si = [si0, si1, si2, si3]
        sgl = [sgl0, sgl1]
        sgr = [sgr0, sgr1]
        ss = [ss0, ss1]
        ci = lax.axis_index("c")
        ti = lax.axis_index("s")
        wid = ci * _NS + ti
        wbase = wid * _GPW
        zeros16 = jnp.zeros((_L,), jnp.float32)

        def zrow(i, carry):
            for k in range(CW // _L):
                wrows[0, i, pl.ds(k * _L, _L)] = zeros16
            return carry
        lax.fori_loop(0, _G, zrow, 0)

        row0 = ti * _RPT
        nch = -(-_RPT // _G)
        for j in range(nch):
            off = min(j * _G, _RPT - _G)
            pltpu.sync_copy(wrows.at[0], acc_sh.at[pl.ds(row0 + off, _G)])
        pltpu.sync_copy(att_hbm, attv)
        plsc.subcore_barrier()

        def issue_idx(g, slot):
            base = (wbase + g) * _G
            pltpu.async_copy(src_hbm.at[pl.ds(base, _G)], sv[slot], si[slot])
            pltpu.async_copy(dst_hbm.at[pl.ds(base, _G)], dv[slot], si[slot])

        def wait_idx(slot):
            pltpu.make_async_copy(src_hbm.at[pl.ds(0, _G)], sv[slot], si[slot]).wait()
            pltpu.make_async_copy(dst_hbm.at[pl.ds(0, _G)], dv[slot], si[slot]).wait()

        def issue_gather(b, slot):
            pltpu.async_copy(l_hbm.at[sv[slot]], lrows.at[b], sgl[b])
            pltpu.async_copy(r_hbm.at[dv[slot]], rrows.at[b], sgr[b])

        def wait_gather(b):
            pltpu.make_async_copy(l_hbm.at[pl.ds(0, _G)], lrows.at[b], sgl[b]).wait()
            pltpu.make_async_copy(r_hbm.at[pl.ds(0, _G)], rrows.at[b], sgr[b]).wait()

        def issue_scatter(b, slot):
            pltpu.async_copy(wrows.at[b], acc_sh.at[dv[slot]], ss[b], add=True)

        def wait_scatter(b, slot):
            pltpu.make_async_copy(wrows.at[b], acc_sh.at[dv[slot]], ss[b]).wait()

        def compute(b):
            attk = [attv[pl.ds(k * _L, _L)] for k in range(NK)]

            @plsc.parallel_loop(0, _G, 1, unroll=unroll)
            def _(e):
                acc = None
                lch = []
                for k in range(NK):
                    vl = lrows[b, e, pl.ds(k * _L, _L)]
                    vr = rrows[b, e, pl.ds(k * _L, _L)]
                    lch.append(vl)
                    s = vl + vr
                    z = jnp.maximum(s, 0.0) + _NEG * jnp.minimum(s, 0.0)
                    t = attk[k] * z
                    acc = t if acc is None else acc + t
                w = jnp.exp(jnp.full((_L,), jnp.sum(acc), jnp.float32))
                for k in range(NK):
                    wrows[b, e, pl.ds(k * _L, _L)] = w * lch[k]
                wrows[b, e, pl.ds(C, _L)] = w

        def stage(g, a, skip_scatter_wait=False):
            R = a % 2
            wait_gather(R)
            if not skip_scatter_wait:
                wait_scatter(R, (a + 2) % 4)
            wait_idx((a + 1) % 4)
            issue_gather(1 - R, (a + 1) % 4)
            compute(R)
            issue_scatter(R, a)
            issue_idx(g + 2, (a + 2) % 4)

        issue_idx(0, 0)
        issue_idx(1, 1)
        wait_idx(0)
        issue_gather(0, 0)
        for a in range(4):
            stage(a, a, skip_scatter_wait=(a < 2))

        def quad(q, carry):
            g = q * 4
            for a in range(4):
                stage(g + a, a)
            return carry
        lax.fori_loop(1, _GPW // 4, quad, 0)

        wait_gather(0)
        wait_scatter(0, 2)
        wait_scatter(1, 3)
        wait_idx(1)

        plsc.subcore_barrier()
        for j in range(nch):
            off = min(j * _G, _RPT - _G)
            sl = pl.ds(row0 + off, _G)
            pltpu.sync_copy(acc_sh.at[sl], out_hbm.at[ci, sl])

    k = pl.kernel(
        body,
        out_type=jax.ShapeDtypeStruct((_NC, _NPAD, CW), jnp.float32),
        mesh=mesh,
        compiler_params=pltpu.CompilerParams(needs_layout_passes=False,
                                             use_tc_tiling_on_sc=False),
        scratch_types=[
            pltpu.VMEM((_G,), jnp.int32),
            pltpu.VMEM((_G,), jnp.int32),
            pltpu.VMEM((_G,), jnp.int32),
            pltpu.VMEM((_G,), jnp.int32),
            pltpu.VMEM((_G,), jnp.int32),
            pltpu.VMEM((_G,), jnp.int32),
            pltpu.VMEM((_G,), jnp.int32),
            pltpu.VMEM((_G,), jnp.int32),
            pltpu.VMEM((2, _G, C), jnp.float32),
            pltpu.VMEM((2, _G, C), jnp.float32),
            pltpu.VMEM((2, _G, CW), jnp.float32),
            pltpu.VMEM((C,), jnp.float32),
            pltpu.VMEM_SHARED((_NPAD, CW), jnp.float32),
            pltpu.SemaphoreType.DMA,
            pltpu.SemaphoreType.DMA,
            pltpu.SemaphoreType.DMA,
            pltpu.SemaphoreType.DMA,
            pltpu.SemaphoreType.DMA,
            pltpu.SemaphoreType.DMA,
            pltpu.SemaphoreType.DMA,
            pltpu.SemaphoreType.DMA,
            pltpu.SemaphoreType.DMA,
            pltpu.SemaphoreType.DMA,
        ],
    )
    return k(l, r_pad, src, dst, att)


def _matmul2(x, Wl, Wr, nb=10):
    N, F = x.shape
    C = Wl.shape[1]

    def body(x_ref, wl_ref, wr_ref, l_ref, r_ref):
        l_ref[...] = x_ref[...] @ wl_ref[...]
        r_ref[...] = x_ref[...] @ wr_ref[...]

    return pl.pallas_call(
        body,
        out_shape=(jax.ShapeDtypeStruct((N, C), x.dtype),
                   jax.ShapeDtypeStruct((N, C), x.dtype)),
        grid=(nb,),
        in_specs=[
            pl.BlockSpec((N // nb, F), lambda i: (i, 0)),
            pl.BlockSpec((F, C), lambda i: (0, 0)),
            pl.BlockSpec((F, C), lambda i: (0, 0)),
        ],
        out_specs=(
            pl.BlockSpec((N // nb, C), lambda i: (i, 0)),
            pl.BlockSpec((N // nb, C), lambda i: (i, 0)),
        ),
    )(x, Wl, Wr)


def _mid(n0, n1, d0, d1, b, Wl, Wr, nb=10):
    N, C = n0.shape
    DW = d0.shape[1]
    K = Wl.shape[1]

    def body(n0_ref, n1_ref, d0_ref, d1_ref, b_ref, wl_ref, wr_ref, l_ref, r_ref):
        num = n0_ref[...] + n1_ref[...]
        den = jnp.sum(d0_ref[...] + d1_ref[...], axis=1, keepdims=True) * (1.0 / 16.0) + 1e-16
        x = num / den + b_ref[...]
        h = jnp.where(x > 0, x, jnp.exp(jnp.minimum(x, 0.0)) - 1.0)
        l_ref[...] = h @ wl_ref[...]
        r_ref[...] = h @ wr_ref[...]

    return pl.pallas_call(
        body,
        out_shape=(jax.ShapeDtypeStruct((N, K), n0.dtype),
                   jax.ShapeDtypeStruct((N, K), n0.dtype)),
        grid=(nb,),
        in_specs=[
            pl.BlockSpec((N // nb, C), lambda i: (i, 0)),
            pl.BlockSpec((N // nb, C), lambda i: (i, 0)),
            pl.BlockSpec((N // nb, DW), lambda i: (i, 0)),
            pl.BlockSpec((N // nb, DW), lambda i: (i, 0)),
            pl.BlockSpec((1, C), lambda i: (0, 0)),
            pl.BlockSpec((C, K), lambda i: (0, 0)),
            pl.BlockSpec((C, K), lambda i: (0, 0)),
        ],
        out_specs=(
            pl.BlockSpec((N // nb, K), lambda i: (i, 0)),
            pl.BlockSpec((N // nb, K), lambda i: (i, 0)),
        ),
    )(n0, n1, d0, d1, b, Wl, Wr)


def _final(m0, m1, d0, d1, b, nb=10):
    N, K = m0.shape
    DW = d0.shape[1]

    def body(m0_ref, m1_ref, d0_ref, d1_ref, b_ref, o_ref):
        num = m0_ref[...] + m1_ref[...]
        den = jnp.sum(d0_ref[...] + d1_ref[...], axis=1, keepdims=True) * (1.0 / 16.0) + 1e-16
        x = num / den + b_ref[...]
        m = jnp.max(x, axis=1, keepdims=True)
        ez = jnp.exp(x - m)
        o_ref[...] = ez / jnp.sum(ez, axis=1, keepdims=True)

    return pl.pallas_call(
        body,
        out_shape=jax.ShapeDtypeStruct((N, K), m0.dtype),
        grid=(nb,),
        in_specs=[
            pl.BlockSpec((N // nb, K), lambda i: (i, 0)),
            pl.BlockSpec((N // nb, K), lambda i: (i, 0)),
            pl.BlockSpec((N // nb, DW), lambda i: (i, 0)),
            pl.BlockSpec((N // nb, DW), lambda i: (i, 0)),
            pl.BlockSpec((1, K), lambda i: (0, 0)),
        ],
        out_specs=pl.BlockSpec((N // nb, K), lambda i: (i, 0)),
    )(m0, m1, d0, d1, b)


def kernel(X, ei_feat, batch, Wl1, Wr1, att1, b1, Wl2, Wr2, att2, b2):
    N = X.shape[0]
    loop = jnp.arange(N, dtype=jnp.int32)
    npad_e = _EALLOC - _E
    src = jnp.concatenate([ei_feat[0].astype(jnp.int32), loop,
                           jnp.zeros((npad_e,), jnp.int32)])
    dst = jnp.concatenate([ei_feat[1].astype(jnp.int32), loop,
                           jnp.full((npad_e,), N, jnp.int32)])

    l1, r1 = _matmul2(X, Wl1, Wr1)
    r1p = jnp.concatenate([r1, jnp.zeros((16, r1.shape[1]), r1.dtype)])
    p1 = _sc_gat_edges(l1, r1p, src, dst, att1, 128, unroll=2)
    l2, r2 = _mid(p1[0, :N, :128], p1[1, :N, :128],
                  p1[0, :N, 128:144], p1[1, :N, 128:144],
                  b1.reshape(1, -1), Wl2, Wr2)

    r2p = jnp.concatenate([r2, jnp.zeros((16, r2.shape[1]), r2.dtype)])
    p2 = _sc_gat_edges(l2, r2p, src, dst, att2, 16)
    return _final(p2[0, :N, :16], p2[1, :N, :16],
                  p2[0, :N, 16:32], p2[1, :N, 16:32],
                  b2.reshape(1, -1))


# speedup vs baseline: 21.4698x; 1.8164x over previous
import functools

import jax
import jax.numpy as jnp
from jax import lax
from jax.experimental import pallas as pl
from jax.experimental.pallas import tpu as pltpu
from jax.experimental.pallas import tpu_sc as plsc

_N = 10000
_NEG = 0.2
_NC = 2
_NS = 16
_L = 16
_NW = _NC * _NS
_G = 48
_E = 330000
_GPW = 216
_EPAD = _NW * _GPW * _G
_EALLOC = _EPAD + 2 * _G
_NPAD = 10016
_RPT = _NPAD // _NS


def _sc_gat_edges(l, r_pad, src, dst, att, C, unroll=4):
    CW = C + 16
    NK = C // _L
    mesh = plsc.VectorSubcoreMesh(core_axis_name="c", subcore_axis_name="s")

    def body(l_hbm, r_hbm, src_hbm, dst_hbm, att_hbm, out_hbm,
             s0, s1, s2, s3, d0, d1, d2, d3,
             lrows, rrows, wrows, attv, acc_sh,
             si0, si1, si2, si3, sgl0, sgl1, sgr0, sgr1, ss0, ss1):
        sv = [s0, s1, s2, s3]
        dv = [d0, d1, d2, d3]
        si = [si0, si1, si2, si3]
        sgl = [sgl0, sgl1]
        sgr = [sgr0, sgr1]
        ss = [ss0, ss1]
        ci = lax.axis_index("c")
        ti = lax.axis_index("s")
        wid = ci * _NS + ti
        wbase = wid * _GPW
        zeros16 = jnp.zeros((_L,), jnp.float32)

        def zrow(i, carry):
            for k in range(CW // _L):
                wrows[0, i, pl.ds(k * _L, _L)] = zeros16
            return carry
        lax.fori_loop(0, _G, zrow, 0)

        row0 = ti * _RPT
        nch = -(-_RPT // _G)
        for j in range(nch):
            off = min(j * _G, _RPT - _G)
            pltpu.sync_copy(wrows.at[0], acc_sh.at[pl.ds(row0 + off, _G)])
        pltpu.sync_copy(att_hbm, attv)
        plsc.subcore_barrier()

        def issue_idx(g, slot):
            base = (wbase + g) * _G
            pltpu.async_copy(src_hbm.at[pl.ds(base, _G)], sv[slot], si[slot])
            pltpu.async_copy(dst_hbm.at[pl.ds(base, _G)], dv[slot], si[slot])

        def wait_idx(slot):
            pltpu.make_async_copy(src_hbm.at[pl.ds(0, _G)], sv[slot], si[slot]).wait()
            pltpu.make_async_copy(dst_hbm.at[pl.ds(0, _G)], dv[slot], si[slot]).wait()

        def issue_gather(b, slot):
            pltpu.async_copy(l_hbm.at[sv[slot]], lrows.at[b], sgl[b])
            pltpu.async_copy(r_hbm.at[dv[slot]], rrows.at[b], sgr[b])

        def wait_gather(b):
            pltpu.make_async_copy(l_hbm.at[pl.ds(0, _G)], lrows.at[b], sgl[b]).wait()
            pltpu.make_async_copy(r_hbm.at[pl.ds(0, _G)], rrows.at[b], sgr[b]).wait()

        def issue_scatter(b, slot):
            pltpu.async_copy(wrows.at[b], acc_sh.at[dv[slot]], ss[b], add=True)

        def wait_scatter(b, slot):
            pltpu.make_async_copy(wrows.at[b], acc_sh.at[dv[slot]], ss[b]).wait()

        def compute(b):
            attk = [attv[pl.ds(k * _L, _L)] for k in range(NK)]

            @plsc.parallel_loop(0, _G, 1, unroll=unroll)
            def _(e):
                acc = None
                lch = []
                for k in range(NK):
                    vl = lrows[b, e, pl.ds(k * _L, _L)]
                    vr = rrows[b, e, pl.ds(k * _L, _L)]
                    lch.append(vl)
                    s = vl + vr
                    z = jnp.maximum(s, 0.0) + _NEG * jnp.minimum(s, 0.0)
                    t = attk[k] * z
                    acc = t if acc is None else acc + t
                w = jnp.exp(jnp.full((_L,), jnp.sum(acc), jnp.float32))
                for k in range(NK):
                    wrows[b, e, pl.ds(k * _L, _L)] = w * lch[k]
                wrows[b, e, pl.ds(C, _L)] = w

        def stage(g, a, skip_scatter_wait=False):
            R = a % 2
            wait_gather(R)
            if not skip_scatter_wait:
                wait_scatter(R, (a + 2) % 4)
            wait_idx((a + 1) % 4)
            issue_gather(1 - R, (a + 1) % 4)
            compute(R)
            issue_scatter(R, a)
            issue_idx(g + 2, (a + 2) % 4)

        issue_idx(0, 0)
        issue_idx(1, 1)
        wait_idx(0)
        issue_gather(0, 0)
        for a in range(4):
            stage(a, a, skip_scatter_wait=(a < 2))

        def quad(q, carry):
            g = q * 4
            for a in range(4):
                stage(g + a, a)
            return carry
        lax.fori_loop(1, _GPW // 4, quad, 0)

        wait_gather(0)
        wait_scatter(0, 2)
        wait_scatter(1, 3)
        wait_idx(1)

        plsc.subcore_barrier()
        for j in range(nch):
            off = min(j * _G, _RPT - _G)
            sl = pl.ds(row0 + off, _G)
            pltpu.sync_copy(acc_sh.at[sl], out_hbm.at[ci, sl])

    k = pl.kernel(
        body,
        out_type=jax.ShapeDtypeStruct((_NC, _NPAD, CW), jnp.float32),
        mesh=mesh,
        compiler_params=pltpu.CompilerParams(needs_layout_passes=False,
                                             use_tc_tiling_on_sc=False),
        scratch_types=[
            pltpu.VMEM((_G,), jnp.int32),
            pltpu.VMEM((_G,), jnp.int32),
            pltpu.VMEM((_G,), jnp.int32),
            pltpu.VMEM((_G,), jnp.int32),
            pltpu.VMEM((_G,), jnp.int32),
            pltpu.VMEM((_G,), jnp.int32),
            pltpu.VMEM((_G,), jnp.int32),
            pltpu.VMEM((_G,), jnp.int32),
            pltpu.VMEM((2, _G, C), jnp.float32),
            pltpu.VMEM((2, _G, C), jnp.float32),
            pltpu.VMEM((2, _G, CW), jnp.float32),
            pltpu.VMEM((C,), jnp.float32),
            pltpu.VMEM_SHARED((_NPAD, CW), jnp.float32),
            pltpu.SemaphoreType.DMA,
            pltpu.SemaphoreType.DMA,
            pltpu.SemaphoreType.DMA,
            pltpu.SemaphoreType.DMA,
            pltpu.SemaphoreType.DMA,
            pltpu.SemaphoreType.DMA,
            pltpu.SemaphoreType.DMA,
            pltpu.SemaphoreType.DMA,
            pltpu.SemaphoreType.DMA,
            pltpu.SemaphoreType.DMA,
        ],
    )
    return k(l, r_pad, src, dst, att)


def _matmul2(x, Wl, Wr, nb=10):
    N, F = x.shape
    C = Wl.shape[1]

    def body(x_ref, wl_ref, wr_ref, l_ref, r_ref):
        l_ref[...] = x_ref[...] @ wl_ref[...]
        r_ref[...] = x_ref[...] @ wr_ref[...]

    return pl.pallas_call(
        body,
        out_shape=(jax.ShapeDtypeStruct((N, C), x.dtype),
                   jax.ShapeDtypeStruct((N, C), x.dtype)),
        grid=(nb,),
        in_specs=[
            pl.BlockSpec((N // nb, F), lambda i: (i, 0)),
            pl.BlockSpec((F, C), lambda i: (0, 0)),
            pl.BlockSpec((F, C), lambda i: (0, 0)),
        ],
        out_specs=(
            pl.BlockSpec((N // nb, C), lambda i: (i, 0)),
            pl.BlockSpec((N // nb, C), lambda i: (i, 0)),
        ),
    )(x, Wl, Wr)


def _mid(n0, n1, d0, d1, b, Wl, Wr, nb=10):
    N, C = n0.shape
    DW = d0.shape[1]
    K = Wl.shape[1]

    def body(n0_ref, n1_ref, d0_ref, d1_ref, b_ref, wl_ref, wr_ref, l_ref, r_ref):
        num = n0_ref[...] + n1_ref[...]
        den = jnp.sum(d0_ref[...] + d1_ref[...], axis=1, keepdims=True) * (1.0 / 16.0) + 1e-16
        x = num / den + b_ref[...]
        h = jnp.where(x > 0, x, jnp.exp(jnp.minimum(x, 0.0)) - 1.0)
        l_ref[...] = h @ wl_ref[...]
        r_ref[...] = h @ wr_ref[...]

    return pl.pallas_call(
        body,
        out_shape=(jax.ShapeDtypeStruct((N, K), n0.dtype),
                   jax.ShapeDtypeStruct((N, K), n0.dtype)),
        grid=(nb,),
        in_specs=[
            pl.BlockSpec((N // nb, C), lambda i: (i, 0)),
            pl.BlockSpec((N // nb, C), lambda i: (i, 0)),
            pl.BlockSpec((N // nb, DW), lambda i: (i, 0)),
            pl.BlockSpec((N // nb, DW), lambda i: (i, 0)),
            pl.BlockSpec((1, C), lambda i: (0, 0)),
            pl.BlockSpec((C, K), lambda i: (0, 0)),
            pl.BlockSpec((C, K), lambda i: (0, 0)),
        ],
        out_specs=(
            pl.BlockSpec((N // nb, K), lambda i: (i, 0)),
            pl.BlockSpec((N // nb, K), lambda i: (i, 0)),
        ),
    )(n0, n1, d0, d1, b, Wl, Wr)


def _final(m0, m1, d0, d1, b, nb=10):
    N, K = m0.shape
    DW = d0.shape[1]

    def body(m0_ref, m1_ref, d0_ref, d1_ref, b_ref, o_ref):
        num = m0_ref[...] + m1_ref[...]
        den = jnp.sum(d0_ref[...] + d1_ref[...], axis=1, keepdims=True) * (1.0 / 16.0) + 1e-16
        x = num / den + b_ref[...]
        m = jnp.max(x, axis=1, keepdims=True)
        ez = jnp.exp(x - m)
        o_ref[...] = ez / jnp.sum(ez, axis=1, keepdims=True)

    return pl.pallas_call(
        body,
        out_shape=jax.ShapeDtypeStruct((N, K), m0.dtype),
        grid=(nb,),
        in_specs=[
            pl.BlockSpec((N // nb, K), lambda i: (i, 0)),
            pl.BlockSpec((N // nb, K), lambda i: (i, 0)),
            pl.BlockSpec((N // nb, DW), lambda i: (i, 0)),
            pl.BlockSpec((N // nb, DW), lambda i: (i, 0)),
            pl.BlockSpec((1, K), lambda i: (0, 0)),
        ],
        out_specs=pl.BlockSpec((N // nb, K), lambda i: (i, 0)),
    )(m0, m1, d0, d1, b)


def kernel(X, ei_feat, batch, Wl1, Wr1, att1, b1, Wl2, Wr2, att2, b2):
    N = X.shape[0]
    loop = jnp.arange(N, dtype=jnp.int32)
    npad_e = _EALLOC - _E
    src = jnp.concatenate([ei_feat[0].astype(jnp.int32), loop,
                           jnp.zeros((npad_e,), jnp.int32)])
    dst = jnp.concatenate([ei_feat[1].astype(jnp.int32), loop,
                           N + (jnp.arange(npad_e, dtype=jnp.int32) % 16)])

    l1, r1 = _matmul2(X, Wl1, Wr1)
    r1p = jnp.concatenate([r1, jnp.zeros((16, r1.shape[1]), r1.dtype)])
    p1 = _sc_gat_edges(l1, r1p, src, dst, att1, 128, unroll=2)
    l2, r2 = _mid(p1[0, :N, :128], p1[1, :N, :128],
                  p1[0, :N, 128:144], p1[1, :N, 128:144],
                  b1.reshape(1, -1), Wl2, Wr2)

    r2p = jnp.concatenate([r2, jnp.zeros((16, r2.shape[1]), r2.dtype)])
    p2 = _sc_gat_edges(l2, r2p, src, dst, att2, 16)
    return _final(p2[0, :N, :16], p2[1, :N, :16],
                  p2[0, :N, 16:32], p2[1, :N, 16:32],
                  b2.reshape(1, -1))
